# baseline (speedup 1.0000x reference)
import functools
import math

import jax
import jax.numpy as jnp
from jax import lax
from jax.experimental import pallas as pl
from jax.experimental.pallas import tpu as pltpu
from jax.experimental.pallas import tpu_sc as plsc

_D = 1024
_LANES = 16
_MAX_TIMESCALE = 10000.0
_SCALE = math.sqrt(_D)

_NC = 2
_NS = 16
_NW = _NC * _NS

_CHUNK = 16
_POS_PER_W = 64
_KPB = _POS_PER_W // _CHUNK


def _pe_table(seq):
    inc = math.log(_MAX_TIMESCALE) / _D
    inv_timescales = jnp.exp(
        jnp.arange(0, _D, 2, dtype=jnp.float32) * -inc)
    position = jnp.arange(0, seq, dtype=jnp.float32)[:, None]
    pe = jnp.zeros((seq, _D), dtype=jnp.float32)
    pe = pe.at[:, 0::2].set(jnp.sin(position * inv_timescales))
    pe = pe.at[:, 1::2].set(jnp.cos(position * inv_timescales))
    return pe


def _sc_embed(table, idx4, pe, batch, seq):
    n_rows = batch * seq
    nchunk = batch * _KPB
    mesh = plsc.VectorSubcoreMesh(core_axis_name="c", subcore_axis_name="s")

    @functools.partial(
        pl.kernel,
        mesh=mesh,
        out_type=jax.ShapeDtypeStruct((n_rows, _D), jnp.float32),
        scratch_types=[
            pltpu.VMEM((nchunk, _CHUNK), jnp.int32),
            pltpu.VMEM((_POS_PER_W, _D), jnp.float32),
            pltpu.VMEM((_CHUNK, _D), jnp.float32),
            pltpu.VMEM((_CHUNK, _D), jnp.float32),
            pltpu.SemaphoreType.DMA,
            pltpu.SemaphoreType.DMA,
            pltpu.SemaphoreType.DMA,
            pltpu.SemaphoreType.DMA,
        ],
    )
    def k(table_hbm, idx_hbm, pe_hbm, out_hbm,
          idx_v, pes, rows0, rows1, g0, g1, w0, w1):
        wid = lax.axis_index("s") * _NC + lax.axis_index("c")
        pos0 = wid * _POS_PER_W
        rows = (rows0, rows1)
        gsem = (g0, g1)
        wsem = (w0, w1)

        pltpu.sync_copy(idx_hbm.at[wid], idx_v)
        pltpu.sync_copy(pe_hbm.at[pl.ds(pos0, _POS_PER_W)], pes)

        gcp = [None, None]
        wcp = [None, None]
        gcp[0] = pltpu.async_copy(table_hbm.at[idx_v.at[0]], rows[0], gsem[0])
        for c in range(nchunk):
            b_el, kp = divmod(c, _KPB)
            buf = c % 2
            nb = 1 - buf
            if c + 1 < nchunk:
                if wcp[nb] is not None:
                    wcp[nb].wait()
                gcp[nb] = pltpu.async_copy(
                    table_hbm.at[idx_v.at[c + 1]], rows[nb], gsem[nb])
            gcp[buf].wait()

            rb = rows[buf]
            k16 = kp * _CHUNK

            @pl.loop(0, _CHUNK)
            def _(r):
                for col in range(_D // _LANES):
                    slc = pl.ds(col * _LANES, _LANES)
                    rb[r, slc] = rb[r, slc] * _SCALE + pes[k16 + r, slc]

            wcp[buf] = pltpu.async_copy(
                rb,
                out_hbm.at[pl.ds(pos0 + b_el * seq + k16, _CHUNK)],
                wsem[buf])
        wcp[0].wait()
        wcp[1].wait()

    return k(table, idx4, pe)


def kernel(x, table):
    batch, seq = x.shape
    n_rows = batch * seq
    assert seq == _NW * _POS_PER_W

    idx4 = (x.reshape(batch, _NW, _KPB, _CHUNK)
             .transpose(1, 0, 2, 3)
             .reshape(_NW, batch * _KPB, _CHUNK))
    pe = _pe_table(seq)
    out = _sc_embed(table, idx4, pe, batch, seq)
    return out.reshape(batch, seq, _D)


# speedup vs baseline: 1.1089x; 1.1089x over previous
import functools
import math

import jax
import jax.numpy as jnp
from jax import lax
from jax.experimental import pallas as pl
from jax.experimental.pallas import tpu as pltpu
from jax.experimental.pallas import tpu_sc as plsc

_D = 1024
_LANES = 16
_MAX_TIMESCALE = 10000.0
_SCALE = math.sqrt(_D)

_NC = 2
_NS = 16
_NW = _NC * _NS

_CHUNK = 32
_NPHASE = 2
_POS_PER_W = _NPHASE * _CHUNK


def _pe_table(seq):
    inc = math.log(_MAX_TIMESCALE) / _D
    inv_timescales = jnp.exp(
        jnp.arange(0, _D, 2, dtype=jnp.float32) * -inc)
    position = jnp.arange(0, seq, dtype=jnp.float32)[:, None]
    pe = jnp.zeros((seq, _D), dtype=jnp.float32)
    pe = pe.at[:, 0::2].set(jnp.sin(position * inv_timescales))
    pe = pe.at[:, 1::2].set(jnp.cos(position * inv_timescales))
    return pe


def _sc_embed(table, idx4, pe, batch, seq):
    n_rows = batch * seq
    nchunk = _NPHASE * batch
    mesh = plsc.VectorSubcoreMesh(core_axis_name="c", subcore_axis_name="s")

    @functools.partial(
        pl.kernel,
        mesh=mesh,
        out_type=jax.ShapeDtypeStruct((n_rows, _D), jnp.float32),
        scratch_types=[
            pltpu.VMEM((nchunk, _CHUNK), jnp.int32),
            pltpu.VMEM((_CHUNK, _D), jnp.float32),
            pltpu.VMEM((_CHUNK, _D), jnp.float32),
            pltpu.VMEM((_CHUNK, _D), jnp.float32),
            pltpu.SemaphoreType.DMA,
            pltpu.SemaphoreType.DMA,
            pltpu.SemaphoreType.DMA,
            pltpu.SemaphoreType.DMA,
        ],
    )
    def k(table_hbm, idx_hbm, pe_hbm, out_hbm,
          idx_v, pes, rows0, rows1, g0, g1, w0, w1):
        wid = lax.axis_index("s") * _NC + lax.axis_index("c")
        pos0 = wid * _POS_PER_W
        rows = (rows0, rows1)
        gsem = (g0, g1)
        wsem = (w0, w1)

        pltpu.sync_copy(idx_hbm.at[wid], idx_v)
        pltpu.sync_copy(pe_hbm.at[pl.ds(pos0, _CHUNK)], pes)

        gcp = [None, None]
        wcp = [None, None]
        gcp[0] = pltpu.async_copy(table_hbm.at[idx_v.at[0]], rows[0], gsem[0])
        for c in range(nchunk):
            ph, b_el = divmod(c, batch)
            buf = c % 2
            nb = 1 - buf
            if c + 1 < nchunk:
                if wcp[nb] is not None:
                    wcp[nb].wait()
                gcp[nb] = pltpu.async_copy(
                    table_hbm.at[idx_v.at[c + 1]], rows[nb], gsem[nb])
            gcp[buf].wait()
            if ph > 0 and b_el == 0:
                pltpu.sync_copy(
                    pe_hbm.at[pl.ds(pos0 + ph * _CHUNK, _CHUNK)], pes)

            rb = rows[buf]

            @pl.loop(0, _CHUNK)
            def _(r):
                for col in range(_D // _LANES):
                    slc = pl.ds(col * _LANES, _LANES)
                    rb[r, slc] = rb[r, slc] * _SCALE + pes[r, slc]

            wcp[buf] = pltpu.async_copy(
                rb,
                out_hbm.at[pl.ds(b_el * seq + pos0 + ph * _CHUNK, _CHUNK)],
                wsem[buf])
        wcp[0].wait()
        wcp[1].wait()

    return k(table, idx4, pe)


def kernel(x, table):
    batch, seq = x.shape
    assert seq == _NW * _POS_PER_W

    idx4 = (x.reshape(batch, _NW, _NPHASE, _CHUNK)
             .transpose(1, 2, 0, 3)
             .reshape(_NW, _NPHASE * batch, _CHUNK))
    pe = _pe_table(seq)
    out = _sc_embed(table, idx4, pe, batch, seq)
    return out.reshape(batch, seq, _D)


# speedup vs baseline: 1.7522x; 1.5802x over previous
import functools
import math

import numpy as np
import jax
import jax.numpy as jnp
from jax import lax
from jax.experimental import pallas as pl
from jax.experimental.pallas import tpu as pltpu
from jax.experimental.pallas import tpu_sc as plsc

_D = 1024
_LANES = 16
_MXLEN = 8192
_MAX_TIMESCALE = 10000.0
_SCALE = math.sqrt(_D)

_NC = 2
_NS = 16
_NW = _NC * _NS

_CHUNK = 32
_NPHASE = 2
_POS_PER_W = _NPHASE * _CHUNK


def _pe_table_np(seq):
    inc = math.log(_MAX_TIMESCALE) / _D
    inv_timescales = np.exp(
        np.arange(0, _D, 2, dtype=np.float32) * -inc).astype(np.float32)
    position = np.arange(0, seq, dtype=np.float32)[:, None]
    pe = np.zeros((seq, _D), dtype=np.float32)
    pe[:, 0::2] = np.sin(position * inv_timescales)
    pe[:, 1::2] = np.cos(position * inv_timescales)
    return pe


_PE = _pe_table_np(_MXLEN)


def _sc_embed(table, idx4, pe, batch, seq):
    n_rows = batch * seq
    nchunk = _NPHASE * batch
    mesh = plsc.VectorSubcoreMesh(core_axis_name="c", subcore_axis_name="s")

    @functools.partial(
        pl.kernel,
        mesh=mesh,
        out_type=jax.ShapeDtypeStruct((n_rows, _D), jnp.float32),
        scratch_types=[
            pltpu.VMEM((nchunk, _CHUNK), jnp.int32),
            pltpu.VMEM((_CHUNK, _D), jnp.float32),
            pltpu.VMEM((_CHUNK, _D), jnp.float32),
            pltpu.VMEM((_CHUNK, _D), jnp.float32),
            pltpu.SemaphoreType.DMA,
            pltpu.SemaphoreType.DMA,
            pltpu.SemaphoreType.DMA,
            pltpu.SemaphoreType.DMA,
        ],
    )
    def k(table_hbm, idx_hbm, pe_hbm, out_hbm,
          idx_v, pes, rows0, rows1, g0, g1, w0, w1):
        wid = lax.axis_index("s") * _NC + lax.axis_index("c")
        pos0 = wid * _POS_PER_W
        rows = (rows0, rows1)
        gsem = (g0, g1)
        wsem = (w0, w1)

        pltpu.sync_copy(idx_hbm.at[wid], idx_v)
        pltpu.sync_copy(pe_hbm.at[pl.ds(pos0, _CHUNK)], pes)

        gcp = [None, None]
        wcp = [None, None]
        gcp[0] = pltpu.async_copy(table_hbm.at[idx_v.at[0]], rows[0], gsem[0])
        for c in range(nchunk):
            ph, b_el = divmod(c, batch)
            buf = c % 2
            nb = 1 - buf
            if c + 1 < nchunk:
                if wcp[nb] is not None:
                    wcp[nb].wait()
                gcp[nb] = pltpu.async_copy(
                    table_hbm.at[idx_v.at[c + 1]], rows[nb], gsem[nb])
            gcp[buf].wait()
            if ph > 0 and b_el == 0:
                pltpu.sync_copy(
                    pe_hbm.at[pl.ds(pos0 + ph * _CHUNK, _CHUNK)], pes)

            rb = rows[buf]

            @pl.loop(0, _CHUNK)
            def _(r):
                for col in range(_D // _LANES):
                    slc = pl.ds(col * _LANES, _LANES)
                    rb[r, slc] = rb[r, slc] * _SCALE + pes[r, slc]

            wcp[buf] = pltpu.async_copy(
                rb,
                out_hbm.at[pl.ds(b_el * seq + pos0 + ph * _CHUNK, _CHUNK)],
                wsem[buf])
        wcp[0].wait()
        wcp[1].wait()

    return k(table, idx4, pe)


def kernel(x, table):
    batch, seq = x.shape
    assert seq == _NW * _POS_PER_W

    idx4 = (x.reshape(batch, _NW, _NPHASE, _CHUNK)
             .transpose(1, 2, 0, 3)
             .reshape(_NW, _NPHASE * batch, _CHUNK))
    pe = jnp.asarray(_PE[:seq])
    out = _sc_embed(table, idx4, pe, batch, seq)
    return out.reshape(batch, seq, _D)


# speedup vs baseline: 1.8299x; 1.0443x over previous
import functools
import math

import numpy as np
import jax
import jax.numpy as jnp
from jax import lax
from jax.experimental import pallas as pl
from jax.experimental.pallas import tpu as pltpu
from jax.experimental.pallas import tpu_sc as plsc

_D = 1024
_MXLEN = 8192
_MAX_TIMESCALE = 10000.0
_SCALE = math.sqrt(_D)

_NC = 2
_NS = 16
_NW = _NC * _NS

_CHUNK = 32
_NCHUNK = 8


def _pe_table_np(seq):
    inc = math.log(_MAX_TIMESCALE) / _D
    inv_timescales = np.exp(
        np.arange(0, _D, 2, dtype=np.float32) * -inc).astype(np.float32)
    position = np.arange(0, seq, dtype=np.float32)[:, None]
    pe = np.zeros((seq, _D), dtype=np.float32)
    pe[:, 0::2] = np.sin(position * inv_timescales)
    pe[:, 1::2] = np.cos(position * inv_timescales)
    return pe


_PE = _pe_table_np(_MXLEN)


def _sc_gather(table, idx3):
    n_rows = _NW * _NCHUNK * _CHUNK
    mesh = plsc.VectorSubcoreMesh(core_axis_name="c", subcore_axis_name="s")

    @functools.partial(
        pl.kernel,
        mesh=mesh,
        out_type=jax.ShapeDtypeStruct((n_rows, _D), jnp.float32),
        scratch_types=[
            pltpu.VMEM((_NCHUNK, _CHUNK), jnp.int32),
            pltpu.VMEM((_CHUNK, _D), jnp.float32),
            pltpu.VMEM((_CHUNK, _D), jnp.float32),
            pltpu.SemaphoreType.DMA,
            pltpu.SemaphoreType.DMA,
            pltpu.SemaphoreType.DMA,
            pltpu.SemaphoreType.DMA,
        ],
    )
    def k(table_hbm, idx_hbm, out_hbm, idx_v, rows0, rows1, g0, g1, w0, w1):
        wid = lax.axis_index("s") * _NC + lax.axis_index("c")
        base = wid * (_NCHUNK * _CHUNK)
        rows = (rows0, rows1)
        gsem = (g0, g1)
        wsem = (w0, w1)
        pltpu.sync_copy(idx_hbm.at[wid], idx_v)
        gcp = [None, None]
        wcp = [None, None]
        gcp[0] = pltpu.async_copy(table_hbm.at[idx_v.at[0]], rows[0], gsem[0])
        for c in range(_NCHUNK):
            b = c % 2
            nb = 1 - b
            if c + 1 < _NCHUNK:
                if wcp[nb] is not None:
                    wcp[nb].wait()
                gcp[nb] = pltpu.async_copy(
                    table_hbm.at[idx_v.at[c + 1]], rows[nb], gsem[nb])
            gcp[b].wait()
            wcp[b] = pltpu.async_copy(
                rows[b], out_hbm.at[pl.ds(base + c * _CHUNK, _CHUNK)], wsem[b])
        wcp[0].wait()
        wcp[1].wait()

    return k(table, idx3)


def _fma_body(g_ref, pe_ref, o_ref):
    o_ref[...] = g_ref[...] * _SCALE + pe_ref[...]


def kernel(x, table):
    batch, seq = x.shape
    n_rows = batch * seq
    assert n_rows == _NW * _NCHUNK * _CHUNK

    idx3 = x.reshape(_NW, _NCHUNK, _CHUNK)
    g = _sc_gather(table, idx3)

    pe = jnp.asarray(_PE[:seq])
    blk = 512
    npe = seq // blk
    out = pl.pallas_call(
        _fma_body,
        grid=(npe, batch),
        in_specs=[
            pl.BlockSpec((blk, _D), lambda i, j: (j * npe + i, 0)),
            pl.BlockSpec((blk, _D), lambda i, j: (i, 0)),
        ],
        out_specs=pl.BlockSpec((blk, _D), lambda i, j: (j * npe + i, 0)),
        out_shape=jax.ShapeDtypeStruct((n_rows, _D), jnp.float32),
    )(g, pe)

    return out.reshape(batch, seq, _D)
